# baseline (speedup 1.0000x reference)
import functools

import jax
import jax.numpy as jnp
from jax import lax
from jax.experimental import pallas as pl
from jax.experimental.pallas import tpu as pltpu

HIDDEN = 768
INTER = 2048
NUM_EXPERTS = 64
N_TOKENS = 32768
CAP = 512
TB = 2048
import numpy as np
_SIGN = np.uint32(0x80000000)


def _router_body(x_ref, gw_ref, keys_ref, max_ref, se_ref, aux_ref,
                 m_scr, s_scr, a_scr):
    i = pl.program_id(0)

    @pl.when(i == 0)
    def _init():
        m_scr[...] = jnp.full_like(m_scr, -jnp.inf)
        s_scr[...] = jnp.zeros_like(s_scr)
        a_scr[...] = jnp.zeros_like(a_scr)

    L = lax.dot_general(gw_ref[...], x_ref[...], (((1,), (1,)), ((), ())),
                        preferred_element_type=jnp.float32)
    m_old = m_scr[...]
    m_new = jnp.maximum(m_old, jnp.max(L, axis=1, keepdims=True))
    s_scr[...] = (s_scr[...] * jnp.exp(m_old - m_new)
                  + jnp.sum(jnp.exp(L - m_new), axis=1, keepdims=True))
    m_scr[...] = m_new
    cm = jnp.max(L, axis=0, keepdims=True)
    lse = cm + jnp.log(jnp.sum(jnp.exp(L - cm), axis=0, keepdims=True))
    a_scr[...] = a_scr[...] + jnp.sum(lse * lse)
    u = lax.bitcast_convert_type(L, jnp.uint32)
    keys_ref[...] = jnp.where(u >= _SIGN, ~u, u | _SIGN)

    @pl.when(i == pl.num_programs(0) - 1)
    def _fin():
        max_ref[...] = m_scr[...]
        se_ref[...] = s_scr[...]
        aux_ref[...] = a_scr[...]


def _router(x, gate_w):
    n_blk = N_TOKENS // TB
    E = NUM_EXPERTS
    return pl.pallas_call(
        _router_body,
        grid=(n_blk,),
        in_specs=[
            pl.BlockSpec((TB, HIDDEN), lambda i: (i, 0)),
            pl.BlockSpec((E, HIDDEN), lambda i: (0, 0)),
        ],
        out_specs=[
            pl.BlockSpec((E, TB), lambda i: (0, i)),
            pl.BlockSpec((E, 1), lambda i: (0, 0)),
            pl.BlockSpec((E, 1), lambda i: (0, 0)),
            pl.BlockSpec((1, 1), lambda i: (0, 0)),
        ],
        out_shape=[
            jax.ShapeDtypeStruct((E, N_TOKENS), jnp.uint32),
            jax.ShapeDtypeStruct((E, 1), jnp.float32),
            jax.ShapeDtypeStruct((E, 1), jnp.float32),
            jax.ShapeDtypeStruct((1, 1), jnp.float32),
        ],
        scratch_shapes=[
            pltpu.VMEM((E, 1), jnp.float32),
            pltpu.VMEM((E, 1), jnp.float32),
            pltpu.VMEM((1, 1), jnp.float32),
        ],
    )(x, gate_w)


_ISPLIT = 2
_IB = INTER // _ISPLIT


def _ffn_body(ein_ref, wg_ref, wu_ref, wd_ref, s_ref, out_ref):
    j = pl.program_id(1)
    xin = ein_ref[0]
    g = lax.dot_general(xin, wg_ref[0], (((1,), (1,)), ((), ())),
                        preferred_element_type=jnp.float32,
                        precision=lax.Precision.DEFAULT)
    up = lax.dot_general(xin, wu_ref[0], (((1,), (1,)), ((), ())),
                         preferred_element_type=jnp.float32,
                         precision=lax.Precision.DEFAULT)
    h = (g * lax.logistic(g)) * up
    o = lax.dot_general(h, wd_ref[0], (((1,), (1,)), ((), ())),
                        preferred_element_type=jnp.float32,
                        precision=lax.Precision.DEFAULT)
    o = o * s_ref[0]

    @pl.when(j == 0)
    def _set():
        out_ref[0] = o

    @pl.when(j != 0)
    def _acc():
        out_ref[0] = out_ref[0] + o


def _ffn(expert_in, gpw, upw, dpw, s3):
    E = expert_in.shape[0]
    return pl.pallas_call(
        _ffn_body,
        grid=(E, _ISPLIT),
        in_specs=[
            pl.BlockSpec((1, CAP, HIDDEN), lambda e, j: (e, 0, 0)),
            pl.BlockSpec((1, _IB, HIDDEN), lambda e, j: (e, j, 0)),
            pl.BlockSpec((1, _IB, HIDDEN), lambda e, j: (e, j, 0)),
            pl.BlockSpec((1, HIDDEN, _IB), lambda e, j: (e, 0, j)),
            pl.BlockSpec((1, CAP, 1), lambda e, j: (e, 0, 0)),
        ],
        out_specs=pl.BlockSpec((1, CAP, HIDDEN), lambda e, j: (e, 0, 0)),
        out_shape=jax.ShapeDtypeStruct((E, CAP, HIDDEN), jnp.float32),
    )(expert_in, gpw, upw, dpw, s3)


_NGRP = 4
_EG = NUM_EXPERTS // _NGRP


from jax.experimental.pallas import tpu_sc as plsc

_LANES = 16
_NSUB = 16
_NCORE = 2
_EPC = NUM_EXPERTS // _NCORE
_EPT = _EPC // _NSUB


def _lane_iota():
    return lax.iota(jnp.int32, _LANES)


def _scalar_at(vec, j):
    return jnp.sum(jnp.where(_lane_iota() == j, vec, jnp.zeros_like(vec)))


def _popcount(mask):
    pc = plsc.all_reduce_population_count(mask)
    return jnp.max(pc)


def _sc_mesh():
    return plsc.VectorSubcoreMesh(core_axis_name="c", subcore_axis_name="s",
                                  num_cores=_NCORE, num_subcores=_NSUB)


def _select_body(keys_hbm, mx_hbm, se_hbm, idx_out, p_out, cnt_out,
                 keycol, hist, bufi, bufk, idx512, p512, mxb, seb, zbuf,
                 cnt_spmem):
    c = lax.axis_index("c")
    s = lax.axis_index("s")
    ii = _lane_iota()

    def _z(i, _):
        zbuf[pl.ds(i * 16, 16)] = jnp.zeros((16,), jnp.float32)
        return 0
    lax.fori_loop(0, 128, _z, 0)

    def _zero_hist(i, _):
        for u in range(4):
            hist[pl.ds(i * 64 + u * 16, 16)] = jnp.zeros((16,), jnp.int32)
        return 0

    pltpu.sync_copy(mx_hbm, mxb.at[pl.ds(0, NUM_EXPERTS)])
    pltpu.sync_copy(se_hbm, seb.at[pl.ds(0, NUM_EXPERTS)])

    def run_expert(cnt_spmem, e, t):
        pltpu.sync_copy(keys_hbm.at[e], keycol)
        mxe = mxb[pl.ds(e, 16)][0]
        se_e = seb[pl.ds(e, 16)][0]

        thr = jnp.uint32(0)
        a_tot = jnp.int32(0)
        for p_i in range(4):
            sh = 24 - 8 * p_i
            lax.fori_loop(0, 64, _zero_hist, 0)
            pre_sh = sh + 8
            tpre = lax.shift_right_logical(thr, jnp.uint32(pre_sh))

            def _scan(i, _, sh=sh, pre_sh=pre_sh, tpre=tpre, p_i=p_i):
                for u in range(4):
                    v = keycol[pl.ds(i * 64 + u * 16, 16)]
                    dig = lax.convert_element_type(
                        lax.shift_right_logical(v, jnp.uint32(sh)),
                        jnp.int32) & 0xFF
                    if p_i == 0:
                        valid = dig >= 0
                    else:
                        valid = lax.shift_right_logical(
                            v, jnp.uint32(pre_sh)) == tpre
                    addr = dig * 16 + ii
                    cur = plsc.load_gather(hist, [addr], mask=valid)
                    plsc.store_scatter(hist, [addr], cur + 1, mask=valid)
                return 0
            lax.fori_loop(0, 512, _scan, 0)

            rem = CAP - a_tot

            def _grp(gi, carry, rem=rem):
                acc, bbin, found, abv = carry
                g = 15 - gi
                tot = jnp.zeros((16,), jnp.int32)
                for l in range(16):
                    tot = tot + plsc.load_gather(hist, [g * 256 + ii * 16 + l])
                rt = lax.rev(tot, (0,))
                cs = plsc.cumsum(rt)
                csa = cs + acc
                m = csa >= rem
                npc = _popcount(m)
                j = 16 - npc
                hit = jnp.logical_and(npc > 0, found == 0)
                bbin = jnp.where(hit, g * 16 + (15 - j), bbin)
                abv = jnp.where(hit, acc + _scalar_at(cs, j) - _scalar_at(rt, j),
                                abv)
                acc = jnp.where(found > 0, acc,
                                jnp.where(hit, acc, acc + _scalar_at(cs, 15)))
                found = jnp.maximum(found, (npc > 0).astype(jnp.int32))
                return acc, bbin, found, abv
            _, bbin, _, abv = lax.fori_loop(
                0, 16, _grp, (jnp.int32(0), jnp.int32(0), jnp.int32(0),
                              jnp.int32(0)))
            thr = thr | lax.shift_left(
                lax.convert_element_type(bbin, jnp.uint32), jnp.uint32(sh))
            a_tot = a_tot + abv

        def _cmp(i, off):
            for u in range(4):
                v = keycol[pl.ds(i * 64 + u * 16, 16)]
                m = jnp.logical_and(v >= thr, off < CAP)
                idxv = i * 64 + u * 16 + ii
                plsc.store_compressed(bufi.at[pl.ds(off, 16)], idxv, mask=m)
                plsc.store_compressed(bufk.at[pl.ds(off, 16)], v, mask=m)
                off = off + _popcount(m)
            return off
        lax.fori_loop(0, 512, _cmp, jnp.int32(0))

        def _pp(j, _):
            kk = bufk[pl.ds(j * 16, 16)]
            u = jnp.where(kk >= _SIGN, kk ^ _SIGN, ~kk)
            lg = plsc.bitcast(u, jnp.float32)
            p512[pl.ds(j * 16, 16)] = jnp.exp(lg - mxe) / se_e
            idx512[pl.ds(j * 16, 16)] = bufi[pl.ds(j * 16, 16)]
            return 0
        lax.fori_loop(0, CAP // 16, _pp, 0)

        pltpu.sync_copy(idx512, idx_out.at[e])
        pltpu.sync_copy(p512, p_out.at[e])
        pltpu.sync_copy(p512, cnt_spmem.at[idx512], add=True)

    pltpu.sync_copy(zbuf, cnt_spmem.at[pl.ds(s * 2048, 2048)])
    plsc.subcore_barrier()
    for t in range(_EPT):
        e = c * _EPC + s * _EPT + t
        run_expert(cnt_spmem, e, t)
    plsc.subcore_barrier()
    pltpu.sync_copy(cnt_spmem.at[pl.ds(s * 2048, 2048)],
                    cnt_out.at[c, pl.ds(s * 2048, 2048)])


def _select(keysT, mx, se):
    E = NUM_EXPERTS
    return pl.kernel(
        _select_body,
        out_type=[
            jax.ShapeDtypeStruct((E, CAP), jnp.int32),
            jax.ShapeDtypeStruct((E, CAP), jnp.float32),
            jax.ShapeDtypeStruct((_NCORE, N_TOKENS), jnp.float32),
        ],
        mesh=_sc_mesh(),
        compiler_params=pltpu.CompilerParams(needs_layout_passes=False),
        scratch_types=[
            pltpu.VMEM((N_TOKENS,), jnp.uint32),
            pltpu.VMEM((4096,), jnp.int32),
            pltpu.VMEM((528,), jnp.int32),
            pltpu.VMEM((528,), jnp.uint32),
            pltpu.VMEM((CAP,), jnp.int32),
            pltpu.VMEM((CAP,), jnp.float32),
            pltpu.VMEM((NUM_EXPERTS + 16,), jnp.float32),
            pltpu.VMEM((NUM_EXPERTS + 16,), jnp.float32),
            pltpu.VMEM((2048,), jnp.float32),
            pltpu.VMEM_SHARED((N_TOKENS,), jnp.float32),
        ],
    )(keysT, mx, se)


def _gather_body(x_hbm, idx_hbm, p_hbm, cnt_hbm, ein_out, s_out,
                 cnt, idxb, pb, sb, rb0, rb1, sem0, sem1):
    c = lax.axis_index("c")
    s = lax.axis_index("s")
    pltpu.sync_copy(cnt_hbm, cnt)
    rbs = (rb0, rb1)
    sems = (sem0, sem1)
    for t in range(_EPT):
        e = c * _EPC + s * _EPT + t
        pltpu.sync_copy(idx_hbm.at[e], idxb)
        pltpu.sync_copy(p_hbm.at[e], pb)

        def _sv(j, _):
            iv = idxb[pl.ds(j * 16, 16)]
            cv = plsc.load_gather(cnt, [iv])
            sb[pl.ds(j * 16, 16)] = pb[pl.ds(j * 16, 16)] / cv
            return 0
        lax.fori_loop(0, CAP // 16, _sv, 0)
        pltpu.sync_copy(sb, s_out.at[e])

        nb = CAP // 32
        descs = []
        for b in range(nb):
            d = pltpu.async_copy(x_hbm.at[idxb.at[pl.ds(b * 32, 32)]],
                                 rbs[b % 2], sems[b % 2])
            descs.append(d)
            if b > 0:
                descs[b - 1].wait()
                pltpu.sync_copy(rbs[(b - 1) % 2],
                                ein_out.at[e].at[pl.ds((b - 1) * 32, 32)])
        descs[nb - 1].wait()
        pltpu.sync_copy(rbs[(nb - 1) % 2],
                        ein_out.at[e].at[pl.ds((nb - 1) * 32, 32)])


def _gather(x, idx, p, counts):
    E = NUM_EXPERTS
    return pl.kernel(
        _gather_body,
        out_type=[
            jax.ShapeDtypeStruct((E, CAP, HIDDEN), jnp.float32),
            jax.ShapeDtypeStruct((E, CAP), jnp.float32),
        ],
        mesh=_sc_mesh(),
        compiler_params=pltpu.CompilerParams(needs_layout_passes=False),
        scratch_types=[
            pltpu.VMEM((N_TOKENS,), jnp.float32),
            pltpu.VMEM((CAP,), jnp.int32),
            pltpu.VMEM((CAP,), jnp.float32),
            pltpu.VMEM((CAP,), jnp.float32),
            pltpu.VMEM((32, HIDDEN), jnp.float32),
            pltpu.VMEM((32, HIDDEN), jnp.float32),
            pltpu.SemaphoreType.DMA,
            pltpu.SemaphoreType.DMA,
        ],
    )(x, idx, p, counts)


_CHUNK = 1024
_NCHUNK = N_TOKENS // _CHUNK


def _scatter_body(w_hbm, idx_hbm, out_hbm, myidx, slotb, locb, slotb32,
                  locb32, rowb, zbuf, sem, chunkbuf):
    c = lax.axis_index("c")
    s = lax.axis_index("s")
    ii = _lane_iota()

    def _zr(r, _):
        def _zc(j, _):
            zbuf[r, pl.ds(j * 16, 16)] = jnp.zeros((16,), jnp.float32)
            return 0
        return lax.fori_loop(0, HIDDEN // 16, _zc, 0)
    lax.fori_loop(0, 32, _zr, 0)
    pltpu.sync_copy(idx_hbm.at[pl.ds(s * 2048, 2048)], myidx)

    for k in range(_NCHUNK // _NCORE):
        chunk = k * _NCORE + c
        base = chunk * _CHUNK
        for q in range(_CHUNK // 16 // 32):
            pltpu.sync_copy(zbuf, chunkbuf.at[pl.ds(s * (_CHUNK // 16) + q * 32, 32)])

        @pl.when(s == 0)
        def _ztrash():
            pltpu.sync_copy(zbuf.at[pl.ds(0, 1)],
                            chunkbuf.at[pl.ds(_CHUNK, 1)])
        plsc.subcore_barrier()

        def _cmp(i, off, chunk=chunk):
            v = myidx[pl.ds(i * 16, 16)]
            m = (v >> 10) == chunk
            slotv = s * 2048 + i * 16 + ii
            locv = v & (_CHUNK - 1)
            plsc.store_compressed(slotb.at[pl.ds(off, 16)], slotv, mask=m)
            plsc.store_compressed(locb.at[pl.ds(off, 16)], locv, mask=m)
            return off + _popcount(m)
        n = lax.fori_loop(0, 128, _cmp, jnp.int32(0))
        slotb[pl.ds(n, 16)] = jnp.zeros((16,), jnp.int32)
        locb[pl.ds(n, 16)] = jnp.full((16,), _CHUNK, jnp.int32)
        slotb[pl.ds(n + 16, 16)] = jnp.zeros((16,), jnp.int32)
        locb[pl.ds(n + 16, 16)] = jnp.full((16,), _CHUNK, jnp.int32)

        nb = (n + 31) >> 5

        def _bat(b, _):
            for h in range(2):
                locb32[pl.ds(h * 16, 16)] = locb[pl.ds(b * 32 + h * 16, 16)]
                slotb32[pl.ds(h * 16, 16)] = slotb[pl.ds(b * 32 + h * 16, 16)]
            pltpu.async_copy(w_hbm.at[slotb32], rowb, sem).wait()
            pltpu.sync_copy(rowb, chunkbuf.at[locb32], add=True)
            return 0
        lax.fori_loop(0, nb, _bat, 0)
        plsc.subcore_barrier()
        for q in range(_CHUNK // 16 // 32):
            pltpu.sync_copy(chunkbuf.at[pl.ds(s * (_CHUNK // 16) + q * 32, 32)],
                            out_hbm.at[pl.ds(base + s * (_CHUNK // 16) + q * 32, 32)])
        plsc.subcore_barrier()


def _scatter(wflat, idxflat):
    return pl.kernel(
        _scatter_body,
        out_type=jax.ShapeDtypeStruct((N_TOKENS, HIDDEN), jnp.float32),
        mesh=_sc_mesh(),
        compiler_params=pltpu.CompilerParams(needs_layout_passes=False),
        scratch_types=[
            pltpu.VMEM((2048,), jnp.int32),
            pltpu.VMEM((2080,), jnp.int32),
            pltpu.VMEM((2080,), jnp.int32),
            pltpu.VMEM((32,), jnp.int32),
            pltpu.VMEM((32,), jnp.int32),
            pltpu.VMEM((32, HIDDEN), jnp.float32),
            pltpu.VMEM((32, HIDDEN), jnp.float32),
            pltpu.SemaphoreType.DMA,
            pltpu.VMEM_SHARED((_CHUNK + 1, HIDDEN), jnp.float32),
        ],
    )(wflat, idxflat)


def _keys_to_logits(keys):
    u = jnp.where(keys >= _SIGN, keys ^ _SIGN, ~keys)
    return lax.bitcast_convert_type(u, jnp.float32)


def kernel(hidden_states, gate_w, gate_proj_w, up_proj_w, down_proj_w):
    B, S, H = hidden_states.shape
    x = hidden_states.reshape(-1, H)
    N = x.shape[0]
    keysT, mx, se, aux_s = _router(x, gate_w)
    aux_loss = aux_s[0, 0] * (0.001 / N)

    top_idx, p, counts2 = _select(keysT, mx[:, 0], se[:, 0])
    counts = jnp.clip(counts2[0] + counts2[1], 1e-9, None)
    expert_in, s = _gather(x, top_idx, p, counts)
    s3 = s[..., None]
    final = jnp.zeros_like(x)
    for g in range(_NGRP):
        sl = slice(g * _EG, (g + 1) * _EG)
        weighted = _ffn(expert_in[sl], gate_proj_w[sl], up_proj_w[sl],
                        down_proj_w[sl], s3[sl])
        final = final.at[top_idx[sl].reshape(-1)].add(
            weighted.reshape(-1, H))
    return final.reshape(B, S, H), aux_loss


# speedup vs baseline: 1.4645x; 1.4645x over previous
import functools

import jax
import jax.numpy as jnp
from jax import lax
from jax.experimental import pallas as pl
from jax.experimental.pallas import tpu as pltpu

HIDDEN = 768
INTER = 2048
NUM_EXPERTS = 64
N_TOKENS = 32768
CAP = 512
TB = 2048
import numpy as np
_SIGN = np.uint32(0x80000000)


def _router_body(x_ref, gw_ref, keys_ref, max_ref, se_ref, aux_ref,
                 m_scr, s_scr, a_scr):
    i = pl.program_id(0)

    @pl.when(i == 0)
    def _init():
        m_scr[...] = jnp.full_like(m_scr, -jnp.inf)
        s_scr[...] = jnp.zeros_like(s_scr)
        a_scr[...] = jnp.zeros_like(a_scr)

    L = lax.dot_general(gw_ref[...], x_ref[...], (((1,), (1,)), ((), ())),
                        preferred_element_type=jnp.float32)
    m_old = m_scr[...]
    m_new = jnp.maximum(m_old, jnp.max(L, axis=1, keepdims=True))
    s_scr[...] = (s_scr[...] * jnp.exp(m_old - m_new)
                  + jnp.sum(jnp.exp(L - m_new), axis=1, keepdims=True))
    m_scr[...] = m_new
    cm = jnp.max(L, axis=0, keepdims=True)
    lse = cm + jnp.log(jnp.sum(jnp.exp(L - cm), axis=0, keepdims=True))
    a_scr[...] = a_scr[...] + jnp.sum(lse * lse)
    u = lax.bitcast_convert_type(L, jnp.uint32)
    keys_ref[...] = jnp.where(u >= _SIGN, ~u, u | _SIGN)

    @pl.when(i == pl.num_programs(0) - 1)
    def _fin():
        max_ref[...] = m_scr[...]
        se_ref[...] = s_scr[...]
        aux_ref[...] = a_scr[...]


def _router(x, gate_w):
    n_blk = N_TOKENS // TB
    E = NUM_EXPERTS
    return pl.pallas_call(
        _router_body,
        grid=(n_blk,),
        in_specs=[
            pl.BlockSpec((TB, HIDDEN), lambda i: (i, 0)),
            pl.BlockSpec((E, HIDDEN), lambda i: (0, 0)),
        ],
        out_specs=[
            pl.BlockSpec((E, TB), lambda i: (0, i)),
            pl.BlockSpec((E, 1), lambda i: (0, 0)),
            pl.BlockSpec((E, 1), lambda i: (0, 0)),
            pl.BlockSpec((1, 1), lambda i: (0, 0)),
        ],
        out_shape=[
            jax.ShapeDtypeStruct((E, N_TOKENS), jnp.uint32),
            jax.ShapeDtypeStruct((E, 1), jnp.float32),
            jax.ShapeDtypeStruct((E, 1), jnp.float32),
            jax.ShapeDtypeStruct((1, 1), jnp.float32),
        ],
        scratch_shapes=[
            pltpu.VMEM((E, 1), jnp.float32),
            pltpu.VMEM((E, 1), jnp.float32),
            pltpu.VMEM((1, 1), jnp.float32),
        ],
    )(x, gate_w)


_ISPLIT = 2
_IB = INTER // _ISPLIT


def _ffn_body(ein_ref, wg_ref, wu_ref, wd_ref, s_ref, out_ref):
    j = pl.program_id(1)
    xin = ein_ref[0]
    g = lax.dot_general(xin, wg_ref[0], (((1,), (1,)), ((), ())),
                        preferred_element_type=jnp.float32,
                        precision=lax.Precision.DEFAULT)
    up = lax.dot_general(xin, wu_ref[0], (((1,), (1,)), ((), ())),
                         preferred_element_type=jnp.float32,
                         precision=lax.Precision.DEFAULT)
    h = (g * lax.logistic(g)) * up
    o = lax.dot_general(h, wd_ref[0], (((1,), (1,)), ((), ())),
                        preferred_element_type=jnp.float32,
                        precision=lax.Precision.DEFAULT)
    o = o * s_ref[0]

    @pl.when(j == 0)
    def _set():
        out_ref[0] = o

    @pl.when(j != 0)
    def _acc():
        out_ref[0] = out_ref[0] + o


def _ffn(expert_in, gpw, upw, dpw, s3):
    E = expert_in.shape[0]
    return pl.pallas_call(
        _ffn_body,
        grid=(E, _ISPLIT),
        in_specs=[
            pl.BlockSpec((1, CAP, HIDDEN), lambda e, j: (e, 0, 0)),
            pl.BlockSpec((1, _IB, HIDDEN), lambda e, j: (e, j, 0)),
            pl.BlockSpec((1, _IB, HIDDEN), lambda e, j: (e, j, 0)),
            pl.BlockSpec((1, HIDDEN, _IB), lambda e, j: (e, 0, j)),
            pl.BlockSpec((1, CAP, 1), lambda e, j: (e, 0, 0)),
        ],
        out_specs=pl.BlockSpec((1, CAP, HIDDEN), lambda e, j: (e, 0, 0)),
        out_shape=jax.ShapeDtypeStruct((E, CAP, HIDDEN), jnp.float32),
    )(expert_in, gpw, upw, dpw, s3)


_NGRP = 4
_EG = NUM_EXPERTS // _NGRP


from jax.experimental.pallas import tpu_sc as plsc

_LANES = 16
_NSUB = 16
_NCORE = 2
_EPC = NUM_EXPERTS // _NCORE
_EPT = _EPC // _NSUB


def _lane_iota():
    return lax.iota(jnp.int32, _LANES)


def _scalar_at(vec, j):
    return jnp.sum(jnp.where(_lane_iota() == j, vec, jnp.zeros_like(vec)))


def _popcount(mask):
    pc = plsc.all_reduce_population_count(mask)
    return jnp.max(pc)


def _sc_mesh():
    return plsc.VectorSubcoreMesh(core_axis_name="c", subcore_axis_name="s",
                                  num_cores=_NCORE, num_subcores=_NSUB)


def _select_body(keys_hbm, mx_hbm, se_hbm, idx_out, p_out, cnt_out,
                 keycol, hist, bufi, bufk, idx512, p512, mxb, seb, zbuf,
                 cnt_spmem):
    c = lax.axis_index("c")
    s = lax.axis_index("s")
    ii = _lane_iota()

    def _z(i, _):
        zbuf[pl.ds(i * 16, 16)] = jnp.zeros((16,), jnp.float32)
        return 0
    lax.fori_loop(0, 128, _z, 0)

    def _zero_hist(i, _):
        for u in range(4):
            hist[pl.ds(i * 64 + u * 16, 16)] = jnp.zeros((16,), jnp.int32)
        return 0

    pltpu.sync_copy(mx_hbm, mxb.at[pl.ds(0, NUM_EXPERTS)])
    pltpu.sync_copy(se_hbm, seb.at[pl.ds(0, NUM_EXPERTS)])

    def run_expert(cnt_spmem, e, t):
        pltpu.sync_copy(keys_hbm.at[e], keycol)
        mxe = mxb[pl.ds(e, 16)][0]
        se_e = seb[pl.ds(e, 16)][0]

        thr = jnp.uint32(0)
        a_tot = jnp.int32(0)
        for p_i in range(4):
            sh = 24 - 8 * p_i
            lax.fori_loop(0, 64, _zero_hist, 0)
            pre_sh = sh + 8
            tpre = lax.shift_right_logical(thr, jnp.uint32(pre_sh))

            def _scan(i, _, sh=sh, pre_sh=pre_sh, tpre=tpre, p_i=p_i):
                for u in range(4):
                    v = keycol[pl.ds(i * 64 + u * 16, 16)]
                    dig = lax.convert_element_type(
                        lax.shift_right_logical(v, jnp.uint32(sh)),
                        jnp.int32) & 0xFF
                    if p_i == 0:
                        valid = dig >= 0
                    else:
                        valid = lax.shift_right_logical(
                            v, jnp.uint32(pre_sh)) == tpre
                    addr = dig * 16 + ii
                    cur = plsc.load_gather(hist, [addr], mask=valid)
                    plsc.store_scatter(hist, [addr], cur + 1, mask=valid)
                return 0
            lax.fori_loop(0, 512, _scan, 0)

            rem = CAP - a_tot

            def _grp(gi, carry, rem=rem):
                acc, bbin, found, abv = carry
                g = 15 - gi
                tot = jnp.zeros((16,), jnp.int32)
                for l in range(16):
                    tot = tot + plsc.load_gather(hist, [g * 256 + ii * 16 + l])
                rt = lax.rev(tot, (0,))
                cs = plsc.cumsum(rt)
                csa = cs + acc
                m = csa >= rem
                npc = _popcount(m)
                j = 16 - npc
                hit = jnp.logical_and(npc > 0, found == 0)
                bbin = jnp.where(hit, g * 16 + (15 - j), bbin)
                abv = jnp.where(hit, acc + _scalar_at(cs, j) - _scalar_at(rt, j),
                                abv)
                acc = jnp.where(found > 0, acc,
                                jnp.where(hit, acc, acc + _scalar_at(cs, 15)))
                found = jnp.maximum(found, (npc > 0).astype(jnp.int32))
                return acc, bbin, found, abv
            _, bbin, _, abv = lax.fori_loop(
                0, 16, _grp, (jnp.int32(0), jnp.int32(0), jnp.int32(0),
                              jnp.int32(0)))
            thr = thr | lax.shift_left(
                lax.convert_element_type(bbin, jnp.uint32), jnp.uint32(sh))
            a_tot = a_tot + abv

        def _cmp(i, off):
            for u in range(4):
                v = keycol[pl.ds(i * 64 + u * 16, 16)]
                m = jnp.logical_and(v >= thr, off < CAP)
                idxv = i * 64 + u * 16 + ii
                plsc.store_compressed(bufi.at[pl.ds(off, 16)], idxv, mask=m)
                plsc.store_compressed(bufk.at[pl.ds(off, 16)], v, mask=m)
                off = off + _popcount(m)
            return off
        lax.fori_loop(0, 512, _cmp, jnp.int32(0))

        def _pp(j, _):
            kk = bufk[pl.ds(j * 16, 16)]
            u = jnp.where(kk >= _SIGN, kk ^ _SIGN, ~kk)
            lg = plsc.bitcast(u, jnp.float32)
            p512[pl.ds(j * 16, 16)] = jnp.exp(lg - mxe) / se_e
            idx512[pl.ds(j * 16, 16)] = bufi[pl.ds(j * 16, 16)]
            return 0
        lax.fori_loop(0, CAP // 16, _pp, 0)

        pltpu.sync_copy(idx512, idx_out.at[e])
        pltpu.sync_copy(p512, p_out.at[e])
        pltpu.sync_copy(p512, cnt_spmem.at[idx512], add=True)

    pltpu.sync_copy(zbuf, cnt_spmem.at[pl.ds(s * 2048, 2048)])
    plsc.subcore_barrier()
    for t in range(_EPT):
        e = c * _EPC + s * _EPT + t
        run_expert(cnt_spmem, e, t)
    plsc.subcore_barrier()
    pltpu.sync_copy(cnt_spmem.at[pl.ds(s * 2048, 2048)],
                    cnt_out.at[c, pl.ds(s * 2048, 2048)])


def _select(keysT, mx, se):
    E = NUM_EXPERTS
    return pl.kernel(
        _select_body,
        out_type=[
            jax.ShapeDtypeStruct((E, CAP), jnp.int32),
            jax.ShapeDtypeStruct((E, CAP), jnp.float32),
            jax.ShapeDtypeStruct((_NCORE, N_TOKENS), jnp.float32),
        ],
        mesh=_sc_mesh(),
        compiler_params=pltpu.CompilerParams(needs_layout_passes=False),
        scratch_types=[
            pltpu.VMEM((N_TOKENS,), jnp.uint32),
            pltpu.VMEM((4096,), jnp.int32),
            pltpu.VMEM((528,), jnp.int32),
            pltpu.VMEM((528,), jnp.uint32),
            pltpu.VMEM((CAP,), jnp.int32),
            pltpu.VMEM((CAP,), jnp.float32),
            pltpu.VMEM((NUM_EXPERTS + 16,), jnp.float32),
            pltpu.VMEM((NUM_EXPERTS + 16,), jnp.float32),
            pltpu.VMEM((2048,), jnp.float32),
            pltpu.VMEM_SHARED((N_TOKENS,), jnp.float32),
        ],
    )(keysT, mx, se)


def _gather_body(x_hbm, idx_hbm, p_hbm, cnt_hbm, ein_out, s_out,
                 cnt, idxb, pb, sb, rb0, rb1, sem0, sem1):
    c = lax.axis_index("c")
    s = lax.axis_index("s")
    pltpu.sync_copy(cnt_hbm, cnt)
    rbs = (rb0, rb1)
    sems = (sem0, sem1)
    for t in range(_EPT):
        e = c * _EPC + s * _EPT + t
        pltpu.sync_copy(idx_hbm.at[e], idxb)
        pltpu.sync_copy(p_hbm.at[e], pb)

        def _sv(j, _):
            iv = idxb[pl.ds(j * 16, 16)]
            cv = plsc.load_gather(cnt, [iv])
            sb[pl.ds(j * 16, 16)] = pb[pl.ds(j * 16, 16)] / cv
            return 0
        lax.fori_loop(0, CAP // 16, _sv, 0)
        pltpu.sync_copy(sb, s_out.at[e])

        nb = CAP // 32
        descs = []
        for b in range(nb):
            d = pltpu.async_copy(x_hbm.at[idxb.at[pl.ds(b * 32, 32)]],
                                 rbs[b % 2], sems[b % 2])
            descs.append(d)
            if b > 0:
                descs[b - 1].wait()
                pltpu.sync_copy(rbs[(b - 1) % 2],
                                ein_out.at[e].at[pl.ds((b - 1) * 32, 32)])
        descs[nb - 1].wait()
        pltpu.sync_copy(rbs[(nb - 1) % 2],
                        ein_out.at[e].at[pl.ds((nb - 1) * 32, 32)])


def _gather(x, idx, p, counts):
    E = NUM_EXPERTS
    return pl.kernel(
        _gather_body,
        out_type=[
            jax.ShapeDtypeStruct((E, CAP, HIDDEN), jnp.float32),
            jax.ShapeDtypeStruct((E, CAP), jnp.float32),
        ],
        mesh=_sc_mesh(),
        compiler_params=pltpu.CompilerParams(needs_layout_passes=False),
        scratch_types=[
            pltpu.VMEM((N_TOKENS,), jnp.float32),
            pltpu.VMEM((CAP,), jnp.int32),
            pltpu.VMEM((CAP,), jnp.float32),
            pltpu.VMEM((CAP,), jnp.float32),
            pltpu.VMEM((32, HIDDEN), jnp.float32),
            pltpu.VMEM((32, HIDDEN), jnp.float32),
            pltpu.SemaphoreType.DMA,
            pltpu.SemaphoreType.DMA,
        ],
    )(x, idx, p, counts)


_CHUNK = 1024
_NCHUNK = N_TOKENS // _CHUNK


def _scatter_body(w_hbm, idx_hbm, out_hbm, myidx, slotb, locb, slotb32,
                  locb32, rowb, zbuf, sem, chunkbuf):
    c = lax.axis_index("c")
    s = lax.axis_index("s")
    ii = _lane_iota()

    def _zr(r, _):
        def _zc(j, _):
            zbuf[r, pl.ds(j * 16, 16)] = jnp.zeros((16,), jnp.float32)
            return 0
        return lax.fori_loop(0, HIDDEN // 16, _zc, 0)
    lax.fori_loop(0, 32, _zr, 0)
    pltpu.sync_copy(idx_hbm.at[pl.ds(s * 2048, 2048)], myidx)

    for k in range(_NCHUNK // _NCORE):
        chunk = k * _NCORE + c
        base = chunk * _CHUNK
        for q in range(_CHUNK // 16 // 32):
            pltpu.sync_copy(zbuf, chunkbuf.at[pl.ds(s * (_CHUNK // 16) + q * 32, 32)])

        @pl.when(s == 0)
        def _ztrash():
            pltpu.sync_copy(zbuf.at[pl.ds(0, 1)],
                            chunkbuf.at[pl.ds(_CHUNK, 1)])
        plsc.subcore_barrier()

        def _cmp(i, off, chunk=chunk):
            v = myidx[pl.ds(i * 16, 16)]
            m = (v >> 10) == chunk
            slotv = s * 2048 + i * 16 + ii
            locv = v & (_CHUNK - 1)
            plsc.store_compressed(slotb.at[pl.ds(off, 16)], slotv, mask=m)
            plsc.store_compressed(locb.at[pl.ds(off, 16)], locv, mask=m)
            return off + _popcount(m)
        n = lax.fori_loop(0, 128, _cmp, jnp.int32(0))
        slotb[pl.ds(n, 16)] = jnp.zeros((16,), jnp.int32)
        locb[pl.ds(n, 16)] = jnp.full((16,), _CHUNK, jnp.int32)
        slotb[pl.ds(n + 16, 16)] = jnp.zeros((16,), jnp.int32)
        locb[pl.ds(n + 16, 16)] = jnp.full((16,), _CHUNK, jnp.int32)

        nb = (n + 31) >> 5

        def _bat(b, _):
            for h in range(2):
                locb32[pl.ds(h * 16, 16)] = locb[pl.ds(b * 32 + h * 16, 16)]
                slotb32[pl.ds(h * 16, 16)] = slotb[pl.ds(b * 32 + h * 16, 16)]
            pltpu.async_copy(w_hbm.at[slotb32], rowb, sem).wait()
            pltpu.sync_copy(rowb, chunkbuf.at[locb32], add=True)
            return 0
        lax.fori_loop(0, nb, _bat, 0)
        plsc.subcore_barrier()
        for q in range(_CHUNK // 16 // 32):
            pltpu.sync_copy(chunkbuf.at[pl.ds(s * (_CHUNK // 16) + q * 32, 32)],
                            out_hbm.at[pl.ds(base + s * (_CHUNK // 16) + q * 32, 32)])
        plsc.subcore_barrier()


def _scatter(wflat, idxflat):
    return pl.kernel(
        _scatter_body,
        out_type=jax.ShapeDtypeStruct((N_TOKENS, HIDDEN), jnp.float32),
        mesh=_sc_mesh(),
        compiler_params=pltpu.CompilerParams(needs_layout_passes=False),
        scratch_types=[
            pltpu.VMEM((2048,), jnp.int32),
            pltpu.VMEM((2080,), jnp.int32),
            pltpu.VMEM((2080,), jnp.int32),
            pltpu.VMEM((32,), jnp.int32),
            pltpu.VMEM((32,), jnp.int32),
            pltpu.VMEM((32, HIDDEN), jnp.float32),
            pltpu.VMEM((32, HIDDEN), jnp.float32),
            pltpu.SemaphoreType.DMA,
            pltpu.VMEM_SHARED((_CHUNK + 1, HIDDEN), jnp.float32),
        ],
    )(wflat, idxflat)


def _keys_to_logits(keys):
    u = jnp.where(keys >= _SIGN, keys ^ _SIGN, ~keys)
    return lax.bitcast_convert_type(u, jnp.float32)


def kernel(hidden_states, gate_w, gate_proj_w, up_proj_w, down_proj_w):
    B, S, H = hidden_states.shape
    x = hidden_states.reshape(-1, H)
    N = x.shape[0]
    keysT, mx, se, aux_s = _router(x, gate_w)
    aux_loss = aux_s[0, 0] * (0.001 / N)

    top_idx, p, counts2 = _select(keysT, mx[:, 0], se[:, 0])
    counts = jnp.clip(counts2[0] + counts2[1], 1e-9, None)
    expert_in, s = _gather(x, top_idx, p, counts)
    weighted = _ffn(expert_in, gate_proj_w, up_proj_w, down_proj_w,
                    s[..., None])
    final = jnp.zeros_like(x).at[top_idx.reshape(-1)].add(
        weighted.reshape(-1, H))
    return final.reshape(B, S, H), aux_loss


# speedup vs baseline: 1.5233x; 1.0401x over previous
import functools

import jax
import jax.numpy as jnp
from jax import lax
from jax.experimental import pallas as pl
from jax.experimental.pallas import tpu as pltpu

HIDDEN = 768
INTER = 2048
NUM_EXPERTS = 64
N_TOKENS = 32768
CAP = 512
TB = 2048
import numpy as np
_SIGN = np.uint32(0x80000000)


def _router_body(x_ref, gw_ref, keys_ref, max_ref, se_ref, aux_ref,
                 m_scr, s_scr, a_scr):
    i = pl.program_id(0)

    @pl.when(i == 0)
    def _init():
        m_scr[...] = jnp.full_like(m_scr, -jnp.inf)
        s_scr[...] = jnp.zeros_like(s_scr)
        a_scr[...] = jnp.zeros_like(a_scr)

    L = lax.dot_general(gw_ref[...], x_ref[...], (((1,), (1,)), ((), ())),
                        preferred_element_type=jnp.float32)
    m_old = m_scr[...]
    m_new = jnp.maximum(m_old, jnp.max(L, axis=1, keepdims=True))
    s_scr[...] = (s_scr[...] * jnp.exp(m_old - m_new)
                  + jnp.sum(jnp.exp(L - m_new), axis=1, keepdims=True))
    m_scr[...] = m_new
    cm = jnp.max(L, axis=0, keepdims=True)
    lse = cm + jnp.log(jnp.sum(jnp.exp(L - cm), axis=0, keepdims=True))
    a_scr[...] = a_scr[...] + jnp.sum(lse * lse)
    u = lax.bitcast_convert_type(L, jnp.uint32)
    keys_ref[...] = jnp.where(u >= _SIGN, ~u, u | _SIGN)

    @pl.when(i == pl.num_programs(0) - 1)
    def _fin():
        max_ref[...] = m_scr[...]
        se_ref[...] = s_scr[...]
        aux_ref[...] = a_scr[...]


def _router(x, gate_w):
    n_blk = N_TOKENS // TB
    E = NUM_EXPERTS
    return pl.pallas_call(
        _router_body,
        grid=(n_blk,),
        in_specs=[
            pl.BlockSpec((TB, HIDDEN), lambda i: (i, 0)),
            pl.BlockSpec((E, HIDDEN), lambda i: (0, 0)),
        ],
        out_specs=[
            pl.BlockSpec((E, TB), lambda i: (0, i)),
            pl.BlockSpec((E, 1), lambda i: (0, 0)),
            pl.BlockSpec((E, 1), lambda i: (0, 0)),
            pl.BlockSpec((1, 1), lambda i: (0, 0)),
        ],
        out_shape=[
            jax.ShapeDtypeStruct((E, N_TOKENS), jnp.uint32),
            jax.ShapeDtypeStruct((E, 1), jnp.float32),
            jax.ShapeDtypeStruct((E, 1), jnp.float32),
            jax.ShapeDtypeStruct((1, 1), jnp.float32),
        ],
        scratch_shapes=[
            pltpu.VMEM((E, 1), jnp.float32),
            pltpu.VMEM((E, 1), jnp.float32),
            pltpu.VMEM((1, 1), jnp.float32),
        ],
    )(x, gate_w)


_ISPLIT = 2
_IB = INTER // _ISPLIT


def _ffn_body(ein_ref, wg_ref, wu_ref, wd_ref, s_ref, out_ref):
    j = pl.program_id(1)
    xin = ein_ref[0]
    g = lax.dot_general(xin, wg_ref[0], (((1,), (1,)), ((), ())),
                        preferred_element_type=jnp.float32,
                        precision=lax.Precision.DEFAULT)
    up = lax.dot_general(xin, wu_ref[0], (((1,), (1,)), ((), ())),
                         preferred_element_type=jnp.float32,
                         precision=lax.Precision.DEFAULT)
    h = (g * lax.logistic(g)) * up
    o = lax.dot_general(h, wd_ref[0], (((1,), (1,)), ((), ())),
                        preferred_element_type=jnp.float32,
                        precision=lax.Precision.DEFAULT)
    o = o * s_ref[0]

    @pl.when(j == 0)
    def _set():
        out_ref[0] = o

    @pl.when(j != 0)
    def _acc():
        out_ref[0] = out_ref[0] + o


def _ffn(expert_in, gpw, upw, dpw, s3):
    E = expert_in.shape[0]
    return pl.pallas_call(
        _ffn_body,
        grid=(E, _ISPLIT),
        in_specs=[
            pl.BlockSpec((1, CAP, HIDDEN), lambda e, j: (e, 0, 0)),
            pl.BlockSpec((1, _IB, HIDDEN), lambda e, j: (e, j, 0)),
            pl.BlockSpec((1, _IB, HIDDEN), lambda e, j: (e, j, 0)),
            pl.BlockSpec((1, HIDDEN, _IB), lambda e, j: (e, 0, j)),
            pl.BlockSpec((1, CAP, 1), lambda e, j: (e, 0, 0)),
        ],
        out_specs=pl.BlockSpec((1, CAP, HIDDEN), lambda e, j: (e, 0, 0)),
        out_shape=jax.ShapeDtypeStruct((E, CAP, HIDDEN), jnp.float32),
    )(expert_in, gpw, upw, dpw, s3)


_NGRP = 4
_EG = NUM_EXPERTS // _NGRP


from jax.experimental.pallas import tpu_sc as plsc

_LANES = 16
_NSUB = 16
_NCORE = 2
_EPC = NUM_EXPERTS // _NCORE
_EPT = _EPC // _NSUB


def _lane_iota():
    return lax.iota(jnp.int32, _LANES)


def _scalar_at(vec, j):
    return jnp.sum(jnp.where(_lane_iota() == j, vec, jnp.zeros_like(vec)))


def _popcount(mask):
    pc = plsc.all_reduce_population_count(mask)
    return jnp.max(pc)


def _sc_mesh():
    return plsc.VectorSubcoreMesh(core_axis_name="c", subcore_axis_name="s",
                                  num_cores=_NCORE, num_subcores=_NSUB)


def _select_body(keys_hbm, mx_hbm, se_hbm, idx_out, p_out, cnt_out,
                 keycol, hist, bufi, bufk, idx512, p512, mxb, seb, zbuf,
                 cnt_spmem):
    c = lax.axis_index("c")
    s = lax.axis_index("s")
    ii = _lane_iota()

    def _z(i, _):
        zbuf[pl.ds(i * 16, 16)] = jnp.zeros((16,), jnp.float32)
        return 0
    lax.fori_loop(0, 128, _z, 0)

    def _zero_hist(i, _):
        for u in range(4):
            hist[pl.ds(i * 64 + u * 16, 16)] = jnp.zeros((16,), jnp.int32)
        return 0

    pltpu.sync_copy(mx_hbm, mxb.at[pl.ds(0, NUM_EXPERTS)])
    pltpu.sync_copy(se_hbm, seb.at[pl.ds(0, NUM_EXPERTS)])

    def run_expert(cnt_spmem, e, t):
        pltpu.sync_copy(keys_hbm.at[e], keycol)
        mxe = mxb[pl.ds(e, 16)][0]
        se_e = seb[pl.ds(e, 16)][0]

        thr = jnp.uint32(0)
        a_tot = jnp.int32(0)
        for p_i in range(4):
            sh = 24 - 8 * p_i
            lax.fori_loop(0, 64, _zero_hist, 0)
            pre_sh = sh + 8
            tpre = lax.shift_right_logical(thr, jnp.uint32(pre_sh))

            def _scan(i, _, sh=sh, pre_sh=pre_sh, tpre=tpre, p_i=p_i):
                for u in range(4):
                    v = keycol[pl.ds(i * 64 + u * 16, 16)]
                    dig = lax.convert_element_type(
                        lax.shift_right_logical(v, jnp.uint32(sh)),
                        jnp.int32) & 0xFF
                    if p_i == 0:
                        valid = dig >= 0
                    else:
                        valid = lax.shift_right_logical(
                            v, jnp.uint32(pre_sh)) == tpre
                    addr = dig * 16 + ii
                    plsc.addupdate_scatter(hist, [addr],
                                           jnp.ones((16,), jnp.int32),
                                           mask=valid)
                return 0
            lax.fori_loop(0, 512, _scan, 0)

            rem = CAP - a_tot

            def _grp(gi, carry, rem=rem):
                acc, bbin, found, abv = carry
                g = 15 - gi
                tot = jnp.zeros((16,), jnp.int32)
                for l in range(16):
                    tot = tot + plsc.load_gather(hist, [g * 256 + ii * 16 + l])
                rt = lax.rev(tot, (0,))
                cs = plsc.cumsum(rt)
                csa = cs + acc
                m = csa >= rem
                npc = _popcount(m)
                j = 16 - npc
                hit = jnp.logical_and(npc > 0, found == 0)
                bbin = jnp.where(hit, g * 16 + (15 - j), bbin)
                abv = jnp.where(hit, acc + _scalar_at(cs, j) - _scalar_at(rt, j),
                                abv)
                acc = jnp.where(found > 0, acc,
                                jnp.where(hit, acc, acc + _scalar_at(cs, 15)))
                found = jnp.maximum(found, (npc > 0).astype(jnp.int32))
                return acc, bbin, found, abv
            _, bbin, _, abv = lax.fori_loop(
                0, 16, _grp, (jnp.int32(0), jnp.int32(0), jnp.int32(0),
                              jnp.int32(0)))
            thr = thr | lax.shift_left(
                lax.convert_element_type(bbin, jnp.uint32), jnp.uint32(sh))
            a_tot = a_tot + abv

        def _cmp(i, off):
            for u in range(4):
                v = keycol[pl.ds(i * 64 + u * 16, 16)]
                m = jnp.logical_and(v >= thr, off < CAP)
                idxv = i * 64 + u * 16 + ii
                plsc.store_compressed(bufi.at[pl.ds(off, 16)], idxv, mask=m)
                plsc.store_compressed(bufk.at[pl.ds(off, 16)], v, mask=m)
                off = off + _popcount(m)
            return off
        lax.fori_loop(0, 512, _cmp, jnp.int32(0))

        def _pp(j, _):
            kk = bufk[pl.ds(j * 16, 16)]
            u = jnp.where(kk >= _SIGN, kk ^ _SIGN, ~kk)
            lg = plsc.bitcast(u, jnp.float32)
            p512[pl.ds(j * 16, 16)] = jnp.exp(lg - mxe) / se_e
            idx512[pl.ds(j * 16, 16)] = bufi[pl.ds(j * 16, 16)]
            return 0
        lax.fori_loop(0, CAP // 16, _pp, 0)

        pltpu.sync_copy(idx512, idx_out.at[e])
        pltpu.sync_copy(p512, p_out.at[e])
        pltpu.sync_copy(p512, cnt_spmem.at[idx512], add=True)

    pltpu.sync_copy(zbuf, cnt_spmem.at[pl.ds(s * 2048, 2048)])
    plsc.subcore_barrier()
    for t in range(_EPT):
        e = c * _EPC + s * _EPT + t
        run_expert(cnt_spmem, e, t)
    plsc.subcore_barrier()
    pltpu.sync_copy(cnt_spmem.at[pl.ds(s * 2048, 2048)],
                    cnt_out.at[c, pl.ds(s * 2048, 2048)])


def _select(keysT, mx, se):
    E = NUM_EXPERTS
    return pl.kernel(
        _select_body,
        out_type=[
            jax.ShapeDtypeStruct((E, CAP), jnp.int32),
            jax.ShapeDtypeStruct((E, CAP), jnp.float32),
            jax.ShapeDtypeStruct((_NCORE, N_TOKENS), jnp.float32),
        ],
        mesh=_sc_mesh(),
        compiler_params=pltpu.CompilerParams(needs_layout_passes=False),
        scratch_types=[
            pltpu.VMEM((N_TOKENS,), jnp.uint32),
            pltpu.VMEM((4096,), jnp.int32),
            pltpu.VMEM((528,), jnp.int32),
            pltpu.VMEM((528,), jnp.uint32),
            pltpu.VMEM((CAP,), jnp.int32),
            pltpu.VMEM((CAP,), jnp.float32),
            pltpu.VMEM((NUM_EXPERTS + 16,), jnp.float32),
            pltpu.VMEM((NUM_EXPERTS + 16,), jnp.float32),
            pltpu.VMEM((2048,), jnp.float32),
            pltpu.VMEM_SHARED((N_TOKENS,), jnp.float32),
        ],
    )(keysT, mx, se)


def _gather_body(x_hbm, idx_hbm, p_hbm, cnt_hbm, ein_out, s_out,
                 cnt, idxb, pb, sb, rb0, rb1, sem0, sem1):
    c = lax.axis_index("c")
    s = lax.axis_index("s")
    pltpu.sync_copy(cnt_hbm, cnt)
    rbs = (rb0, rb1)
    sems = (sem0, sem1)
    for t in range(_EPT):
        e = c * _EPC + s * _EPT + t
        pltpu.sync_copy(idx_hbm.at[e], idxb)
        pltpu.sync_copy(p_hbm.at[e], pb)

        def _sv(j, _):
            iv = idxb[pl.ds(j * 16, 16)]
            cv = plsc.load_gather(cnt, [iv])
            sb[pl.ds(j * 16, 16)] = pb[pl.ds(j * 16, 16)] / cv
            return 0
        lax.fori_loop(0, CAP // 16, _sv, 0)
        pltpu.sync_copy(sb, s_out.at[e])

        nb = CAP // 32
        descs = []
        for b in range(nb):
            d = pltpu.async_copy(x_hbm.at[idxb.at[pl.ds(b * 32, 32)]],
                                 rbs[b % 2], sems[b % 2])
            descs.append(d)
            if b > 0:
                descs[b - 1].wait()
                pltpu.sync_copy(rbs[(b - 1) % 2],
                                ein_out.at[e].at[pl.ds((b - 1) * 32, 32)])
        descs[nb - 1].wait()
        pltpu.sync_copy(rbs[(nb - 1) % 2],
                        ein_out.at[e].at[pl.ds((nb - 1) * 32, 32)])


def _gather(x, idx, p, counts):
    E = NUM_EXPERTS
    return pl.kernel(
        _gather_body,
        out_type=[
            jax.ShapeDtypeStruct((E, CAP, HIDDEN), jnp.float32),
            jax.ShapeDtypeStruct((E, CAP), jnp.float32),
        ],
        mesh=_sc_mesh(),
        compiler_params=pltpu.CompilerParams(needs_layout_passes=False),
        scratch_types=[
            pltpu.VMEM((N_TOKENS,), jnp.float32),
            pltpu.VMEM((CAP,), jnp.int32),
            pltpu.VMEM((CAP,), jnp.float32),
            pltpu.VMEM((CAP,), jnp.float32),
            pltpu.VMEM((32, HIDDEN), jnp.float32),
            pltpu.VMEM((32, HIDDEN), jnp.float32),
            pltpu.SemaphoreType.DMA,
            pltpu.SemaphoreType.DMA,
        ],
    )(x, idx, p, counts)


_CHUNK = 1024
_NCHUNK = N_TOKENS // _CHUNK


def _scatter_body(w_hbm, idx_hbm, out_hbm, myidx, slotb, locb, slotb32,
                  locb32, rowb, zbuf, sem, chunkbuf):
    c = lax.axis_index("c")
    s = lax.axis_index("s")
    ii = _lane_iota()

    def _zr(r, _):
        def _zc(j, _):
            zbuf[r, pl.ds(j * 16, 16)] = jnp.zeros((16,), jnp.float32)
            return 0
        return lax.fori_loop(0, HIDDEN // 16, _zc, 0)
    lax.fori_loop(0, 32, _zr, 0)
    pltpu.sync_copy(idx_hbm.at[pl.ds(s * 2048, 2048)], myidx)

    for k in range(_NCHUNK // _NCORE):
        chunk = k * _NCORE + c
        base = chunk * _CHUNK
        for q in range(_CHUNK // 16 // 32):
            pltpu.sync_copy(zbuf, chunkbuf.at[pl.ds(s * (_CHUNK // 16) + q * 32, 32)])

        @pl.when(s == 0)
        def _ztrash():
            pltpu.sync_copy(zbuf.at[pl.ds(0, 1)],
                            chunkbuf.at[pl.ds(_CHUNK, 1)])
        plsc.subcore_barrier()

        def _cmp(i, off, chunk=chunk):
            v = myidx[pl.ds(i * 16, 16)]
            m = (v >> 10) == chunk
            slotv = s * 2048 + i * 16 + ii
            locv = v & (_CHUNK - 1)
            plsc.store_compressed(slotb.at[pl.ds(off, 16)], slotv, mask=m)
            plsc.store_compressed(locb.at[pl.ds(off, 16)], locv, mask=m)
            return off + _popcount(m)
        n = lax.fori_loop(0, 128, _cmp, jnp.int32(0))
        slotb[pl.ds(n, 16)] = jnp.zeros((16,), jnp.int32)
        locb[pl.ds(n, 16)] = jnp.full((16,), _CHUNK, jnp.int32)
        slotb[pl.ds(n + 16, 16)] = jnp.zeros((16,), jnp.int32)
        locb[pl.ds(n + 16, 16)] = jnp.full((16,), _CHUNK, jnp.int32)

        nb = (n + 31) >> 5

        def _bat(b, _):
            for h in range(2):
                locb32[pl.ds(h * 16, 16)] = locb[pl.ds(b * 32 + h * 16, 16)]
                slotb32[pl.ds(h * 16, 16)] = slotb[pl.ds(b * 32 + h * 16, 16)]
            pltpu.async_copy(w_hbm.at[slotb32], rowb, sem).wait()
            pltpu.sync_copy(rowb, chunkbuf.at[locb32], add=True)
            return 0
        lax.fori_loop(0, nb, _bat, 0)
        plsc.subcore_barrier()
        for q in range(_CHUNK // 16 // 32):
            pltpu.sync_copy(chunkbuf.at[pl.ds(s * (_CHUNK // 16) + q * 32, 32)],
                            out_hbm.at[pl.ds(base + s * (_CHUNK // 16) + q * 32, 32)])
        plsc.subcore_barrier()


def _scatter(wflat, idxflat):
    return pl.kernel(
        _scatter_body,
        out_type=jax.ShapeDtypeStruct((N_TOKENS, HIDDEN), jnp.float32),
        mesh=_sc_mesh(),
        compiler_params=pltpu.CompilerParams(needs_layout_passes=False),
        scratch_types=[
            pltpu.VMEM((2048,), jnp.int32),
            pltpu.VMEM((2080,), jnp.int32),
            pltpu.VMEM((2080,), jnp.int32),
            pltpu.VMEM((32,), jnp.int32),
            pltpu.VMEM((32,), jnp.int32),
            pltpu.VMEM((32, HIDDEN), jnp.float32),
            pltpu.VMEM((32, HIDDEN), jnp.float32),
            pltpu.SemaphoreType.DMA,
            pltpu.VMEM_SHARED((_CHUNK + 1, HIDDEN), jnp.float32),
        ],
    )(wflat, idxflat)


def _keys_to_logits(keys):
    u = jnp.where(keys >= _SIGN, keys ^ _SIGN, ~keys)
    return lax.bitcast_convert_type(u, jnp.float32)


def kernel(hidden_states, gate_w, gate_proj_w, up_proj_w, down_proj_w):
    B, S, H = hidden_states.shape
    x = hidden_states.reshape(-1, H)
    N = x.shape[0]
    keysT, mx, se, aux_s = _router(x, gate_w)
    aux_loss = aux_s[0, 0] * (0.001 / N)

    top_idx, p, counts2 = _select(keysT, mx[:, 0], se[:, 0])
    counts = jnp.clip(counts2[0] + counts2[1], 1e-9, None)
    expert_in, s = _gather(x, top_idx, p, counts)
    weighted = _ffn(expert_in, gate_proj_w, up_proj_w, down_proj_w,
                    s[..., None])
    final = jnp.zeros_like(x).at[top_idx.reshape(-1)].add(
        weighted.reshape(-1, H))
    return final.reshape(B, S, H), aux_loss


# speedup vs baseline: 1.5243x; 1.0007x over previous
import jax
import jax.numpy as jnp
from jax import lax
from jax.experimental import pallas as pl
from jax.experimental.pallas import tpu as pltpu

HIDDEN = 768
INTER = 2048
NUM_EXPERTS = 64
N_TOKENS = 32768
CAP = 512
TB = 2048
import numpy as np
_SIGN = np.uint32(0x80000000)


def _router_body(x_ref, gw_ref, keys_ref, max_ref, se_ref, aux_ref,
                 m_scr, s_scr, a_scr):
    i = pl.program_id(0)

    @pl.when(i == 0)
    def _init():
        m_scr[...] = jnp.full_like(m_scr, -jnp.inf)
        s_scr[...] = jnp.zeros_like(s_scr)
        a_scr[...] = jnp.zeros_like(a_scr)

    L = lax.dot_general(gw_ref[...], x_ref[...], (((1,), (1,)), ((), ())),
                        preferred_element_type=jnp.float32)
    m_old = m_scr[...]
    m_new = jnp.maximum(m_old, jnp.max(L, axis=1, keepdims=True))
    s_scr[...] = (s_scr[...] * jnp.exp(m_old - m_new)
                  + jnp.sum(jnp.exp(L - m_new), axis=1, keepdims=True))
    m_scr[...] = m_new
    cm = jnp.max(L, axis=0, keepdims=True)
    lse = cm + jnp.log(jnp.sum(jnp.exp(L - cm), axis=0, keepdims=True))
    a_scr[...] = a_scr[...] + jnp.sum(lse * lse)
    u = lax.bitcast_convert_type(L, jnp.uint32)
    keys_ref[...] = jnp.where(u >= _SIGN, ~u, u | _SIGN)

    @pl.when(i == pl.num_programs(0) - 1)
    def _fin():
        max_ref[...] = m_scr[...]
        se_ref[...] = s_scr[...]
        aux_ref[...] = a_scr[...]


def _router(x, gate_w):
    n_blk = N_TOKENS // TB
    E = NUM_EXPERTS
    return pl.pallas_call(
        _router_body,
        grid=(n_blk,),
        in_specs=[
            pl.BlockSpec((TB, HIDDEN), lambda i: (i, 0)),
            pl.BlockSpec((E, HIDDEN), lambda i: (0, 0)),
        ],
        out_specs=[
            pl.BlockSpec((E, TB), lambda i: (0, i)),
            pl.BlockSpec((E, 1), lambda i: (0, 0)),
            pl.BlockSpec((E, 1), lambda i: (0, 0)),
            pl.BlockSpec((1, 1), lambda i: (0, 0)),
        ],
        out_shape=[
            jax.ShapeDtypeStruct((E, N_TOKENS), jnp.uint32),
            jax.ShapeDtypeStruct((E, 1), jnp.float32),
            jax.ShapeDtypeStruct((E, 1), jnp.float32),
            jax.ShapeDtypeStruct((1, 1), jnp.float32),
        ],
        scratch_shapes=[
            pltpu.VMEM((E, 1), jnp.float32),
            pltpu.VMEM((E, 1), jnp.float32),
            pltpu.VMEM((1, 1), jnp.float32),
        ],
    )(x, gate_w)


_ISPLIT = 2
_IB = INTER // _ISPLIT


def _ffn_body(ein_ref, wg_ref, wu_ref, wd_ref, s_ref, out_ref):
    j = pl.program_id(1)
    xin = ein_ref[0]
    g = lax.dot_general(xin, wg_ref[0], (((1,), (1,)), ((), ())),
                        preferred_element_type=jnp.float32,
                        precision=lax.Precision.DEFAULT)
    up = lax.dot_general(xin, wu_ref[0], (((1,), (1,)), ((), ())),
                         preferred_element_type=jnp.float32,
                         precision=lax.Precision.DEFAULT)
    h = (g * lax.logistic(g)) * up
    o = lax.dot_general(h, wd_ref[0], (((1,), (1,)), ((), ())),
                        preferred_element_type=jnp.float32,
                        precision=lax.Precision.DEFAULT)
    o = o * s_ref[0]

    @pl.when(j == 0)
    def _set():
        out_ref[0] = o

    @pl.when(j != 0)
    def _acc():
        out_ref[0] = out_ref[0] + o


def _ffn(expert_in, gpw, upw, dpw, s3):
    E = expert_in.shape[0]
    return pl.pallas_call(
        _ffn_body,
        grid=(E, _ISPLIT),
        in_specs=[
            pl.BlockSpec((1, CAP, HIDDEN), lambda e, j: (e, 0, 0)),
            pl.BlockSpec((1, _IB, HIDDEN), lambda e, j: (e, j, 0)),
            pl.BlockSpec((1, _IB, HIDDEN), lambda e, j: (e, j, 0)),
            pl.BlockSpec((1, HIDDEN, _IB), lambda e, j: (e, 0, j)),
            pl.BlockSpec((1, CAP, 1), lambda e, j: (e, 0, 0)),
        ],
        out_specs=pl.BlockSpec((1, CAP, HIDDEN), lambda e, j: (e, 0, 0)),
        out_shape=jax.ShapeDtypeStruct((E, CAP, HIDDEN), jnp.float32),
    )(expert_in, gpw, upw, dpw, s3)


from jax.experimental.pallas import tpu_sc as plsc

_LANES = 16
_NSUB = 16
_NCORE = 2
_EPC = NUM_EXPERTS // _NCORE
_EPT = _EPC // _NSUB


def _lane_iota():
    return lax.iota(jnp.int32, _LANES)


def _scalar_at(vec, j):
    return jnp.sum(jnp.where(_lane_iota() == j, vec, jnp.zeros_like(vec)))


def _popcount(mask):
    pc = plsc.all_reduce_population_count(mask)
    return jnp.max(pc)


def _sc_mesh():
    return plsc.VectorSubcoreMesh(core_axis_name="c", subcore_axis_name="s",
                                  num_cores=_NCORE, num_subcores=_NSUB)


def _select_body(keys_hbm, mx_hbm, se_hbm, idx_out, p_out, cnt_out,
                 keycol, hist, bufi, bufk, idx512, p512, mxb, seb, zbuf,
                 cnt_spmem):
    c = lax.axis_index("c")
    s = lax.axis_index("s")
    ii = _lane_iota()

    def _z(i, _):
        zbuf[pl.ds(i * 16, 16)] = jnp.zeros((16,), jnp.float32)
        return 0
    lax.fori_loop(0, 128, _z, 0)

    def _zero_hist(i, _):
        for u in range(4):
            hist[pl.ds(i * 64 + u * 16, 16)] = jnp.zeros((16,), jnp.int32)
        return 0

    pltpu.sync_copy(mx_hbm, mxb.at[pl.ds(0, NUM_EXPERTS)])
    pltpu.sync_copy(se_hbm, seb.at[pl.ds(0, NUM_EXPERTS)])

    def run_expert(cnt_spmem, e, t):
        pltpu.sync_copy(keys_hbm.at[e], keycol)
        mxe = mxb[pl.ds(e, 16)][0]
        se_e = seb[pl.ds(e, 16)][0]

        thr = jnp.uint32(0)
        a_tot = jnp.int32(0)
        for p_i in range(4):
            sh = 24 - 8 * p_i
            lax.fori_loop(0, 64, _zero_hist, 0)
            pre_sh = sh + 8
            tpre = lax.shift_right_logical(thr, jnp.uint32(pre_sh))

            def _scan(i, _, sh=sh, pre_sh=pre_sh, tpre=tpre, p_i=p_i):
                for u in range(4):
                    v = keycol[pl.ds(i * 64 + u * 16, 16)]
                    dig = lax.convert_element_type(
                        lax.shift_right_logical(v, jnp.uint32(sh)),
                        jnp.int32) & 0xFF
                    if p_i == 0:
                        valid = dig >= 0
                    else:
                        valid = lax.shift_right_logical(
                            v, jnp.uint32(pre_sh)) == tpre
                    addr = dig * 16 + ii
                    plsc.addupdate_scatter(hist, [addr],
                                           jnp.ones((16,), jnp.int32),
                                           mask=valid)
                return 0
            lax.fori_loop(0, 512, _scan, 0)

            rem = CAP - a_tot

            def _grp(gi, carry, rem=rem):
                acc, bbin, found, abv = carry
                g = 15 - gi
                tot = jnp.zeros((16,), jnp.int32)
                for l in range(16):
                    tot = tot + plsc.load_gather(hist, [g * 256 + ii * 16 + l])
                rt = lax.rev(tot, (0,))
                cs = plsc.cumsum(rt)
                csa = cs + acc
                m = csa >= rem
                npc = _popcount(m)
                j = 16 - npc
                hit = jnp.logical_and(npc > 0, found == 0)
                bbin = jnp.where(hit, g * 16 + (15 - j), bbin)
                abv = jnp.where(hit, acc + _scalar_at(cs, j) - _scalar_at(rt, j),
                                abv)
                acc = jnp.where(found > 0, acc,
                                jnp.where(hit, acc, acc + _scalar_at(cs, 15)))
                found = jnp.maximum(found, (npc > 0).astype(jnp.int32))
                return acc, bbin, found, abv
            _, bbin, _, abv = lax.fori_loop(
                0, 16, _grp, (jnp.int32(0), jnp.int32(0), jnp.int32(0),
                              jnp.int32(0)))
            thr = thr | lax.shift_left(
                lax.convert_element_type(bbin, jnp.uint32), jnp.uint32(sh))
            a_tot = a_tot + abv

        def _cmp(i, off):
            for u in range(4):
                v = keycol[pl.ds(i * 64 + u * 16, 16)]
                m = jnp.logical_and(v >= thr, off < CAP)
                idxv = i * 64 + u * 16 + ii
                plsc.store_compressed(bufi.at[pl.ds(off, 16)], idxv, mask=m)
                plsc.store_compressed(bufk.at[pl.ds(off, 16)], v, mask=m)
                off = off + _popcount(m)
            return off
        lax.fori_loop(0, 512, _cmp, jnp.int32(0))

        def _pp(j, _):
            kk = bufk[pl.ds(j * 16, 16)]
            u = jnp.where(kk >= _SIGN, kk ^ _SIGN, ~kk)
            lg = plsc.bitcast(u, jnp.float32)
            p512[pl.ds(j * 16, 16)] = jnp.exp(lg - mxe) / se_e
            idx512[pl.ds(j * 16, 16)] = bufi[pl.ds(j * 16, 16)]
            return 0
        lax.fori_loop(0, CAP // 16, _pp, 0)

        pltpu.sync_copy(idx512, idx_out.at[e])
        pltpu.sync_copy(p512, p_out.at[e])
        pltpu.sync_copy(p512, cnt_spmem.at[idx512], add=True)

    pltpu.sync_copy(zbuf, cnt_spmem.at[pl.ds(s * 2048, 2048)])
    plsc.subcore_barrier()
    for t in range(_EPT):
        e = c * _EPC + s * _EPT + t
        run_expert(cnt_spmem, e, t)
    plsc.subcore_barrier()
    pltpu.sync_copy(cnt_spmem.at[pl.ds(s * 2048, 2048)],
                    cnt_out.at[c, pl.ds(s * 2048, 2048)])


def _select(keysT, mx, se):
    E = NUM_EXPERTS
    return pl.kernel(
        _select_body,
        out_type=[
            jax.ShapeDtypeStruct((E, CAP), jnp.int32),
            jax.ShapeDtypeStruct((E, CAP), jnp.float32),
            jax.ShapeDtypeStruct((_NCORE, N_TOKENS), jnp.float32),
        ],
        mesh=_sc_mesh(),
        compiler_params=pltpu.CompilerParams(needs_layout_passes=False),
        scratch_types=[
            pltpu.VMEM((N_TOKENS,), jnp.uint32),
            pltpu.VMEM((4096,), jnp.int32),
            pltpu.VMEM((528,), jnp.int32),
            pltpu.VMEM((528,), jnp.uint32),
            pltpu.VMEM((CAP,), jnp.int32),
            pltpu.VMEM((CAP,), jnp.float32),
            pltpu.VMEM((NUM_EXPERTS + 16,), jnp.float32),
            pltpu.VMEM((NUM_EXPERTS + 16,), jnp.float32),
            pltpu.VMEM((2048,), jnp.float32),
            pltpu.VMEM_SHARED((N_TOKENS,), jnp.float32),
        ],
    )(keysT, mx, se)


def _gather_body(x_hbm, idx_hbm, p_hbm, cnt_hbm, ein_out, s_out,
                 cnt, idxb, pb, sb, rb0, rb1, sem0, sem1):
    c = lax.axis_index("c")
    s = lax.axis_index("s")
    pltpu.sync_copy(cnt_hbm, cnt)
    rbs = (rb0, rb1)
    sems = (sem0, sem1)
    for t in range(_EPT):
        e = c * _EPC + s * _EPT + t
        pltpu.sync_copy(idx_hbm.at[e], idxb)
        pltpu.sync_copy(p_hbm.at[e], pb)

        def _sv(j, _):
            iv = idxb[pl.ds(j * 16, 16)]
            cv = plsc.load_gather(cnt, [iv])
            sb[pl.ds(j * 16, 16)] = pb[pl.ds(j * 16, 16)] / cv
            return 0
        lax.fori_loop(0, CAP // 16, _sv, 0)
        pltpu.sync_copy(sb, s_out.at[e])

        nb = CAP // 32
        descs = []
        for b in range(nb):
            d = pltpu.async_copy(x_hbm.at[idxb.at[pl.ds(b * 32, 32)]],
                                 rbs[b % 2], sems[b % 2])
            descs.append(d)
            if b > 0:
                descs[b - 1].wait()
                pltpu.sync_copy(rbs[(b - 1) % 2],
                                ein_out.at[e].at[pl.ds((b - 1) * 32, 32)])
        descs[nb - 1].wait()
        pltpu.sync_copy(rbs[(nb - 1) % 2],
                        ein_out.at[e].at[pl.ds((nb - 1) * 32, 32)])


def _gather(x, idx, p, counts):
    E = NUM_EXPERTS
    return pl.kernel(
        _gather_body,
        out_type=[
            jax.ShapeDtypeStruct((E, CAP, HIDDEN), jnp.float32),
            jax.ShapeDtypeStruct((E, CAP), jnp.float32),
        ],
        mesh=_sc_mesh(),
        compiler_params=pltpu.CompilerParams(needs_layout_passes=False),
        scratch_types=[
            pltpu.VMEM((N_TOKENS,), jnp.float32),
            pltpu.VMEM((CAP,), jnp.int32),
            pltpu.VMEM((CAP,), jnp.float32),
            pltpu.VMEM((CAP,), jnp.float32),
            pltpu.VMEM((32, HIDDEN), jnp.float32),
            pltpu.VMEM((32, HIDDEN), jnp.float32),
            pltpu.SemaphoreType.DMA,
            pltpu.SemaphoreType.DMA,
        ],
    )(x, idx, p, counts)


def _keys_to_logits(keys):
    u = jnp.where(keys >= _SIGN, keys ^ _SIGN, ~keys)
    return lax.bitcast_convert_type(u, jnp.float32)


def kernel(hidden_states, gate_w, gate_proj_w, up_proj_w, down_proj_w):
    B, S, H = hidden_states.shape
    x = hidden_states.reshape(-1, H)
    N = x.shape[0]
    keysT, mx, se, aux_s = _router(x, gate_w)
    aux_loss = aux_s[0, 0] * (0.001 / N)

    top_idx, p, counts2 = _select(keysT, mx[:, 0], se[:, 0])
    counts = jnp.clip(counts2[0] + counts2[1], 1e-9, None)
    expert_in, s = _gather(x, top_idx, p, counts)
    weighted = _ffn(expert_in, gate_proj_w, up_proj_w, down_proj_w,
                    s[..., None])
    final = jnp.zeros_like(x).at[top_idx.reshape(-1)].add(
        weighted.reshape(-1, H))
    return final.reshape(B, S, H), aux_loss


# speedup vs baseline: 1.5245x; 1.0001x over previous
import jax
import jax.numpy as jnp
from jax import lax
from jax.experimental import pallas as pl
from jax.experimental.pallas import tpu as pltpu

HIDDEN = 768
INTER = 2048
NUM_EXPERTS = 64
N_TOKENS = 32768
CAP = 512
TB = 2048
import numpy as np
_SIGN = np.uint32(0x80000000)


def _router_body(x_ref, gw_ref, keys_ref, max_ref, se_ref, aux_ref,
                 m_scr, s_scr, a_scr):
    i = pl.program_id(0)

    @pl.when(i == 0)
    def _init():
        m_scr[...] = jnp.full_like(m_scr, -jnp.inf)
        s_scr[...] = jnp.zeros_like(s_scr)
        a_scr[...] = jnp.zeros_like(a_scr)

    L = lax.dot_general(gw_ref[...], x_ref[...], (((1,), (1,)), ((), ())),
                        preferred_element_type=jnp.float32)
    m_old = m_scr[...]
    m_new = jnp.maximum(m_old, jnp.max(L, axis=1, keepdims=True))
    s_scr[...] = (s_scr[...] * jnp.exp(m_old - m_new)
                  + jnp.sum(jnp.exp(L - m_new), axis=1, keepdims=True))
    m_scr[...] = m_new
    cm = jnp.max(L, axis=0, keepdims=True)
    lse = cm + jnp.log(jnp.sum(jnp.exp(L - cm), axis=0, keepdims=True))
    a_scr[...] = a_scr[...] + jnp.sum(lse * lse)
    u = lax.bitcast_convert_type(L, jnp.uint32)
    keys_ref[...] = jnp.where(u >= _SIGN, ~u, u | _SIGN)

    @pl.when(i == pl.num_programs(0) - 1)
    def _fin():
        max_ref[...] = m_scr[...]
        se_ref[...] = s_scr[...]
        aux_ref[...] = a_scr[...]


def _router(x, gate_w):
    n_blk = N_TOKENS // TB
    E = NUM_EXPERTS
    return pl.pallas_call(
        _router_body,
        grid=(n_blk,),
        in_specs=[
            pl.BlockSpec((TB, HIDDEN), lambda i: (i, 0)),
            pl.BlockSpec((E, HIDDEN), lambda i: (0, 0)),
        ],
        out_specs=[
            pl.BlockSpec((E, TB), lambda i: (0, i)),
            pl.BlockSpec((E, 1), lambda i: (0, 0)),
            pl.BlockSpec((E, 1), lambda i: (0, 0)),
            pl.BlockSpec((1, 1), lambda i: (0, 0)),
        ],
        out_shape=[
            jax.ShapeDtypeStruct((E, N_TOKENS), jnp.uint32),
            jax.ShapeDtypeStruct((E, 1), jnp.float32),
            jax.ShapeDtypeStruct((E, 1), jnp.float32),
            jax.ShapeDtypeStruct((1, 1), jnp.float32),
        ],
        scratch_shapes=[
            pltpu.VMEM((E, 1), jnp.float32),
            pltpu.VMEM((E, 1), jnp.float32),
            pltpu.VMEM((1, 1), jnp.float32),
        ],
    )(x, gate_w)


_ISPLIT = 2
_IB = INTER // _ISPLIT


def _ffn_body(ein_ref, wg_ref, wu_ref, wd_ref, s_ref, out_ref):
    j = pl.program_id(1)
    xin = ein_ref[0]
    g = lax.dot_general(xin, wg_ref[0], (((1,), (1,)), ((), ())),
                        preferred_element_type=jnp.float32,
                        precision=lax.Precision.DEFAULT)
    up = lax.dot_general(xin, wu_ref[0], (((1,), (1,)), ((), ())),
                         preferred_element_type=jnp.float32,
                         precision=lax.Precision.DEFAULT)
    h = (g * lax.logistic(g)) * up
    o = lax.dot_general(h, wd_ref[0], (((1,), (1,)), ((), ())),
                        preferred_element_type=jnp.float32,
                        precision=lax.Precision.DEFAULT)
    o = o * s_ref[0]

    @pl.when(j == 0)
    def _set():
        out_ref[0] = o

    @pl.when(j != 0)
    def _acc():
        out_ref[0] = out_ref[0] + o


def _ffn(expert_in, gpw, upw, dpw, s3):
    E = expert_in.shape[0]
    return pl.pallas_call(
        _ffn_body,
        grid=(E, _ISPLIT),
        in_specs=[
            pl.BlockSpec((1, CAP, HIDDEN), lambda e, j: (e, 0, 0)),
            pl.BlockSpec((1, _IB, HIDDEN), lambda e, j: (e, j, 0)),
            pl.BlockSpec((1, _IB, HIDDEN), lambda e, j: (e, j, 0)),
            pl.BlockSpec((1, HIDDEN, _IB), lambda e, j: (e, 0, j)),
            pl.BlockSpec((1, CAP, 1), lambda e, j: (e, 0, 0)),
        ],
        out_specs=pl.BlockSpec((1, CAP, HIDDEN), lambda e, j: (e, 0, 0)),
        out_shape=jax.ShapeDtypeStruct((E, CAP, HIDDEN), jnp.float32),
    )(expert_in, gpw, upw, dpw, s3)


from jax.experimental.pallas import tpu_sc as plsc

_LANES = 16
_NSUB = 16
_NCORE = 2
_EPC = NUM_EXPERTS // _NCORE
_EPT = _EPC // _NSUB


def _lane_iota():
    return lax.iota(jnp.int32, _LANES)


def _scalar_at(vec, j):
    return jnp.sum(jnp.where(_lane_iota() == j, vec, jnp.zeros_like(vec)))


def _popcount(mask):
    pc = plsc.all_reduce_population_count(mask)
    return jnp.max(pc)


def _sc_mesh():
    return plsc.VectorSubcoreMesh(core_axis_name="c", subcore_axis_name="s",
                                  num_cores=_NCORE, num_subcores=_NSUB)


def _select_body(keys_hbm, mx_hbm, se_hbm, idx_out, p_out, cnt_out,
                 keycol, hist, bufi, bufk, idx512, p512, mxb, seb, zbuf,
                 cnt_spmem):
    c = lax.axis_index("c")
    s = lax.axis_index("s")
    ii = _lane_iota()

    def _z(i, _):
        zbuf[pl.ds(i * 16, 16)] = jnp.zeros((16,), jnp.float32)
        return 0
    lax.fori_loop(0, 128, _z, 0)

    def _zero_hist(i, _):
        for u in range(4):
            hist[pl.ds(i * 64 + u * 16, 16)] = jnp.zeros((16,), jnp.int32)
        return 0

    pltpu.sync_copy(mx_hbm, mxb.at[pl.ds(0, NUM_EXPERTS)])
    pltpu.sync_copy(se_hbm, seb.at[pl.ds(0, NUM_EXPERTS)])

    def run_expert(cnt_spmem, e, t):
        pltpu.sync_copy(keys_hbm.at[e], keycol)
        mxe = mxb[pl.ds(e, 16)][0]
        se_e = seb[pl.ds(e, 16)][0]

        thr = jnp.uint32(0)
        a_tot = jnp.int32(0)
        for p_i in range(4):
            sh = 24 - 8 * p_i
            lax.fori_loop(0, 64, _zero_hist, 0)
            pre_sh = sh + 8
            tpre = lax.shift_right_logical(thr, jnp.uint32(pre_sh))

            def _scan(i, _, sh=sh, pre_sh=pre_sh, tpre=tpre, p_i=p_i):
                for u in range(4):
                    v = keycol[pl.ds(i * 64 + u * 16, 16)]
                    dig = lax.convert_element_type(
                        lax.shift_right_logical(v, jnp.uint32(sh)),
                        jnp.int32) & 0xFF
                    if p_i == 0:
                        valid = dig >= 0
                    else:
                        valid = lax.shift_right_logical(
                            v, jnp.uint32(pre_sh)) == tpre
                    addr = dig * 16 + ii
                    plsc.addupdate_scatter(hist, [addr],
                                           jnp.ones((16,), jnp.int32),
                                           mask=valid)
                return 0
            lax.fori_loop(0, 512, _scan, 0)

            rem = CAP - a_tot

            def _grp(gi, carry, rem=rem):
                acc, bbin, found, abv = carry
                g = 15 - gi
                tot = jnp.zeros((16,), jnp.int32)
                for l in range(16):
                    tot = tot + plsc.load_gather(hist, [g * 256 + ii * 16 + l])
                rt = lax.rev(tot, (0,))
                cs = plsc.cumsum(rt)
                csa = cs + acc
                m = csa >= rem
                npc = _popcount(m)
                j = 16 - npc
                hit = jnp.logical_and(npc > 0, found == 0)
                bbin = jnp.where(hit, g * 16 + (15 - j), bbin)
                abv = jnp.where(hit, acc + _scalar_at(cs, j) - _scalar_at(rt, j),
                                abv)
                acc = jnp.where(found > 0, acc,
                                jnp.where(hit, acc, acc + _scalar_at(cs, 15)))
                found = jnp.maximum(found, (npc > 0).astype(jnp.int32))
                return acc, bbin, found, abv
            _, bbin, _, abv = lax.fori_loop(
                0, 16, _grp, (jnp.int32(0), jnp.int32(0), jnp.int32(0),
                              jnp.int32(0)))
            thr = thr | lax.shift_left(
                lax.convert_element_type(bbin, jnp.uint32), jnp.uint32(sh))
            a_tot = a_tot + abv

        def _cmp(i, off):
            for u in range(4):
                v = keycol[pl.ds(i * 64 + u * 16, 16)]
                m = jnp.logical_and(v >= thr, off < CAP)
                idxv = i * 64 + u * 16 + ii
                plsc.store_compressed(bufi.at[pl.ds(off, 16)], idxv, mask=m)
                plsc.store_compressed(bufk.at[pl.ds(off, 16)], v, mask=m)
                off = off + _popcount(m)
            return off
        lax.fori_loop(0, 512, _cmp, jnp.int32(0))

        def _pp(j, _):
            kk = bufk[pl.ds(j * 16, 16)]
            u = jnp.where(kk >= _SIGN, kk ^ _SIGN, ~kk)
            lg = plsc.bitcast(u, jnp.float32)
            p512[pl.ds(j * 16, 16)] = jnp.exp(lg - mxe) / se_e
            idx512[pl.ds(j * 16, 16)] = bufi[pl.ds(j * 16, 16)]
            return 0
        lax.fori_loop(0, CAP // 16, _pp, 0)

        pltpu.sync_copy(idx512, idx_out.at[e])
        pltpu.sync_copy(p512, p_out.at[e])
        pltpu.sync_copy(p512, cnt_spmem.at[idx512], add=True)

    pltpu.sync_copy(zbuf, cnt_spmem.at[pl.ds(s * 2048, 2048)])
    plsc.subcore_barrier()
    for t in range(_EPT):
        e = c * _EPC + s * _EPT + t
        run_expert(cnt_spmem, e, t)
    plsc.subcore_barrier()
    pltpu.sync_copy(cnt_spmem.at[pl.ds(s * 2048, 2048)],
                    cnt_out.at[c, pl.ds(s * 2048, 2048)])


def _select(keysT, mx, se):
    E = NUM_EXPERTS
    return pl.kernel(
        _select_body,
        out_type=[
            jax.ShapeDtypeStruct((E, CAP), jnp.int32),
            jax.ShapeDtypeStruct((E, CAP), jnp.float32),
            jax.ShapeDtypeStruct((_NCORE, N_TOKENS), jnp.float32),
        ],
        mesh=_sc_mesh(),
        compiler_params=pltpu.CompilerParams(needs_layout_passes=False),
        scratch_types=[
            pltpu.VMEM((N_TOKENS,), jnp.uint32),
            pltpu.VMEM((4096,), jnp.int32),
            pltpu.VMEM((528,), jnp.int32),
            pltpu.VMEM((528,), jnp.uint32),
            pltpu.VMEM((CAP,), jnp.int32),
            pltpu.VMEM((CAP,), jnp.float32),
            pltpu.VMEM((NUM_EXPERTS + 16,), jnp.float32),
            pltpu.VMEM((NUM_EXPERTS + 16,), jnp.float32),
            pltpu.VMEM((2048,), jnp.float32),
            pltpu.VMEM_SHARED((N_TOKENS,), jnp.float32),
        ],
    )(keysT, mx, se)


def _gather_body(x_hbm, idx_hbm, p_hbm, cnt_hbm, ein_out, s_out,
                 cnt, idxb, pb, sb, rb0, rb1, sem0, sem1):
    c = lax.axis_index("c")
    s = lax.axis_index("s")
    pltpu.sync_copy(cnt_hbm, cnt)
    rbs = (rb0, rb1)
    sems = (sem0, sem1)
    for t in range(_EPT):
        e = c * _EPC + s * _EPT + t
        pltpu.sync_copy(idx_hbm.at[e], idxb)
        pltpu.sync_copy(p_hbm.at[e], pb)

        def _sv(j, _):
            iv = idxb[pl.ds(j * 16, 16)]
            cv = plsc.load_gather(cnt, [iv])
            sb[pl.ds(j * 16, 16)] = pb[pl.ds(j * 16, 16)] / cv
            return 0
        lax.fori_loop(0, CAP // 16, _sv, 0)
        pltpu.sync_copy(sb, s_out.at[e])

        nb = CAP // 32
        descs = []
        for b in range(nb):
            d = pltpu.async_copy(x_hbm.at[idxb.at[pl.ds(b * 32, 32)]],
                                 rbs[b % 2], sems[b % 2])
            descs.append(d)
            if b > 0:
                descs[b - 1].wait()
                pltpu.sync_copy(rbs[(b - 1) % 2],
                                ein_out.at[e].at[pl.ds((b - 1) * 32, 32)])
        descs[nb - 1].wait()
        pltpu.sync_copy(rbs[(nb - 1) % 2],
                        ein_out.at[e].at[pl.ds((nb - 1) * 32, 32)])


def _gather(x, idx, p, counts):
    E = NUM_EXPERTS
    return pl.kernel(
        _gather_body,
        out_type=[
            jax.ShapeDtypeStruct((E, CAP, HIDDEN), jnp.float32),
            jax.ShapeDtypeStruct((E, CAP), jnp.float32),
        ],
        mesh=_sc_mesh(),
        compiler_params=pltpu.CompilerParams(needs_layout_passes=False),
        scratch_types=[
            pltpu.VMEM((N_TOKENS,), jnp.float32),
            pltpu.VMEM((CAP,), jnp.int32),
            pltpu.VMEM((CAP,), jnp.float32),
            pltpu.VMEM((CAP,), jnp.float32),
            pltpu.VMEM((32, HIDDEN), jnp.float32),
            pltpu.VMEM((32, HIDDEN), jnp.float32),
            pltpu.SemaphoreType.DMA,
            pltpu.SemaphoreType.DMA,
        ],
    )(x, idx, p, counts)


def kernel(hidden_states, gate_w, gate_proj_w, up_proj_w, down_proj_w):
    B, S, H = hidden_states.shape
    x = hidden_states.reshape(-1, H)
    N = x.shape[0]
    keysT, mx, se, aux_s = _router(x, gate_w)
    aux_loss = aux_s[0, 0] * (0.001 / N)

    top_idx, p, counts2 = _select(keysT, mx[:, 0], se[:, 0])
    counts = jnp.clip(counts2[0] + counts2[1], 1e-9, None)
    expert_in, s = _gather(x, top_idx, p, counts)
    weighted = _ffn(expert_in, gate_proj_w, up_proj_w, down_proj_w,
                    s[..., None])
    final = jnp.zeros_like(x).at[top_idx.reshape(-1)].add(
        weighted.reshape(-1, H))
    return final.reshape(B, S, H), aux_loss
